# Initial kernel scaffold; baseline (speedup 1.0000x reference)
import jax
import jax.numpy as jnp
from jax.experimental import pallas as pl


def kernel(weights, membership, W, b):
    raise NotImplementedError("write your pallas kernel here")


# speedup vs baseline: 490.2711x; 490.2711x over previous
import functools

import jax
import jax.numpy as jnp
from jax import lax
from jax.experimental import pallas as pl
from jax.experimental.pallas import tpu as pltpu
from jax.experimental.pallas import tpu_sc as plsc

B, N_ITEMS, N_SETS, K_SET, D = 256, 2000, 500, 50, 16

NC, NS, L = 2, 16, 16
NW = NC * NS
ROWS_PER_W = B // NW
NBLK = (N_SETS + L - 1) // L
S_PAD = NBLK * L
M_LEN = N_SETS * K_SET
M_PAD = S_PAD * K_SET
OUT_LEN = N_SETS * D


def _sc_body(w_hbm, m_hbm, wv_hbm, bias_hbm, out_hbm,
             w_v, m_v, wv_v, bias_v, out_v):
    cid = lax.axis_index("c")
    sid = lax.axis_index("s")
    wid = sid * NC + cid

    lane = lax.iota(jnp.int32, L)
    lane50 = lane * K_SET
    lane16 = lane * D

    pltpu.sync_copy(wv_hbm, wv_v)
    pltpu.sync_copy(bias_hbm, bias_v)

    z16 = jnp.zeros((L,), jnp.int32)
    for t in range(M_LEN - (M_LEN % L), M_PAD, L):
        m_v[pl.ds(t, L)] = z16

    wv_vec = wv_v[...]
    kb_vec = jnp.float32(K_SET) * bias_v[...]
    wv = [wv_vec[d] for d in range(D)]
    kb = [kb_vec[d] for d in range(D)]

    for r in range(ROWS_PER_W):
        b = wid * ROWS_PER_W + r
        pltpu.sync_copy(w_hbm.at[pl.ds(b * N_ITEMS, N_ITEMS)], w_v)
        pltpu.sync_copy(m_hbm.at[pl.ds(b * M_LEN, M_LEN)],
                        m_v.at[pl.ds(0, M_LEN)])

        def blk_body(i, _):
            addr0 = lane50 + i * (L * K_SET)
            acc = jnp.zeros((L,), jnp.float32)
            for k in range(K_SET):
                iv = plsc.load_gather(m_v, [addr0 + k])
                vv = plsc.load_gather(w_v, [iv])
                acc = acc + vv
            obase = lane16 + i * (L * D)
            for d in range(D):
                plsc.store_scatter(out_v, [obase + d], acc * wv[d] + kb[d])
            return 0

        lax.fori_loop(0, NBLK, blk_body, 0)
        pltpu.sync_copy(out_v.at[pl.ds(0, OUT_LEN)],
                        out_hbm.at[pl.ds(b * OUT_LEN, OUT_LEN)])


@jax.jit
def _mcp_embed(weights, memb_flat, wv, bias):
    run = pl.kernel(
        _sc_body,
        out_type=jax.ShapeDtypeStruct((B * OUT_LEN,), jnp.float32),
        mesh=plsc.VectorSubcoreMesh(core_axis_name="c", subcore_axis_name="s"),
        scratch_types=[
            pltpu.VMEM((N_ITEMS,), jnp.float32),
            pltpu.VMEM((M_PAD,), jnp.int32),
            pltpu.VMEM((D,), jnp.float32),
            pltpu.VMEM((D,), jnp.float32),
            pltpu.VMEM((S_PAD * D,), jnp.float32),
        ],
        compiler_params=pltpu.CompilerParams(needs_layout_passes=False),
    )
    return run(weights, memb_flat, wv, bias)


def kernel(weights, membership, W, b):
    memb_flat = membership.astype(jnp.int32).reshape(B * M_LEN)
    out = _mcp_embed(weights.reshape(B * N_ITEMS), memb_flat, W[:, 0], b)
    return out.reshape(B, N_SETS, D)


# speedup vs baseline: 524.4641x; 1.0697x over previous
import jax
import jax.numpy as jnp
from jax import lax
from jax.experimental import pallas as pl
from jax.experimental.pallas import tpu as pltpu
from jax.experimental.pallas import tpu_sc as plsc

B, N_ITEMS, N_SETS, K_SET, D = 256, 2000, 500, 50, 16

NC, NS, L = 2, 16, 16
NW = NC * NS
ROWS_PER_W = B // NW
NBLK = (N_SETS + L - 1) // L
S_PAD = NBLK * L
M_LEN = N_SETS * K_SET
M_PAD = S_PAD * K_SET
OUT_LEN = N_SETS * D


def _sc_body(w_hbm, m_hbm, wv_hbm, bias_hbm, out_hbm,
             w_v0, w_v1, m_v0, m_v1, o_v0, o_v1, wv_v, bias_v,
             sem_w, sem_m, sem_o):
    cid = lax.axis_index("c")
    sid = lax.axis_index("s")
    wid = sid * NC + cid

    lane = lax.iota(jnp.int32, L)
    lane50 = lane * K_SET
    lane16 = lane * D

    pltpu.sync_copy(wv_hbm, wv_v)
    pltpu.sync_copy(bias_hbm, bias_v)

    z16 = jnp.zeros((L,), jnp.int32)
    for m_v in (m_v0, m_v1):
        for t in range(M_LEN - (M_LEN % L), M_PAD, L):
            m_v[pl.ds(t, L)] = z16

    wv_vec = wv_v[...]
    kb_vec = jnp.float32(K_SET) * bias_v[...]
    wv = [wv_vec[d] for d in range(D)]
    kb = [kb_vec[d] for d in range(D)]

    w_bufs = (w_v0, w_v1)
    m_bufs = (m_v0, m_v1)
    o_bufs = (o_v0, o_v1)

    def row_of(r):
        return wid * ROWS_PER_W + r

    def issue_in(r):
        b = row_of(r)
        dw = pltpu.async_copy(
            w_hbm.at[pl.ds(b * N_ITEMS, N_ITEMS)], w_bufs[r % 2], sem_w)
        dm = pltpu.async_copy(
            m_hbm.at[pl.ds(b * M_LEN, M_LEN)],
            m_bufs[r % 2].at[pl.ds(0, M_LEN)], sem_m)
        return dw, dm

    in_descs = {0: issue_in(0)}
    out_descs = [None, None]

    for r in range(ROWS_PER_W):
        w_v, m_v, o_v = w_bufs[r % 2], m_bufs[r % 2], o_bufs[r % 2]
        dw, dm = in_descs.pop(r)
        dw.wait()
        dm.wait()
        if r + 1 < ROWS_PER_W:
            in_descs[r + 1] = issue_in(r + 1)
        if out_descs[r % 2] is not None:
            out_descs[r % 2].wait()

        def blk_body(i):
            addr0 = lane50 + i * (L * K_SET)
            acc0 = jnp.zeros((L,), jnp.float32)
            acc1 = jnp.zeros((L,), jnp.float32)
            for k in range(0, K_SET, 2):
                iv0 = plsc.load_gather(m_v, [addr0 + k])
                iv1 = plsc.load_gather(m_v, [addr0 + (k + 1)])
                acc0 = acc0 + plsc.load_gather(w_v, [iv0])
                acc1 = acc1 + plsc.load_gather(w_v, [iv1])
            acc = acc0 + acc1
            obase = lane16 + i * (L * D)
            for d in range(D):
                plsc.store_scatter(o_v, [obase + d], acc * wv[d] + kb[d])

        plsc.parallel_loop(0, NBLK, 1, unroll=1)(blk_body)

        out_descs[r % 2] = pltpu.async_copy(
            o_v.at[pl.ds(0, OUT_LEN)],
            out_hbm.at[pl.ds(row_of(r) * OUT_LEN, OUT_LEN)], sem_o)

    out_descs[0].wait()
    out_descs[1].wait()


@jax.jit
def _mcp_embed(weights, memb_flat, wv, bias):
    run = pl.kernel(
        _sc_body,
        out_type=jax.ShapeDtypeStruct((B * OUT_LEN,), jnp.float32),
        mesh=plsc.VectorSubcoreMesh(core_axis_name="c", subcore_axis_name="s"),
        scratch_types=[
            pltpu.VMEM((N_ITEMS,), jnp.float32),
            pltpu.VMEM((N_ITEMS,), jnp.float32),
            pltpu.VMEM((M_PAD,), jnp.int32),
            pltpu.VMEM((M_PAD,), jnp.int32),
            pltpu.VMEM((S_PAD * D,), jnp.float32),
            pltpu.VMEM((S_PAD * D,), jnp.float32),
            pltpu.VMEM((D,), jnp.float32),
            pltpu.VMEM((D,), jnp.float32),
            pltpu.SemaphoreType.DMA,
            pltpu.SemaphoreType.DMA,
            pltpu.SemaphoreType.DMA,
        ],
        compiler_params=pltpu.CompilerParams(needs_layout_passes=False),
    )
    return run(weights, memb_flat, wv, bias)


def kernel(weights, membership, W, b):
    memb_flat = membership.astype(jnp.int32).reshape(B * M_LEN)
    out = _mcp_embed(weights.reshape(B * N_ITEMS), memb_flat, W[:, 0], b)
    return out.reshape(B, N_SETS, D)
